# Initial kernel scaffold; baseline (speedup 1.0000x reference)
import jax
import jax.numpy as jnp
from jax.experimental import pallas as pl


def kernel(sentence_token, emb, Wih_f, Whh_f, bih_f, bhh_f, Wih_b, Whh_b, bih_b, bhh_b, Wout, bout):
    raise NotImplementedError("write your pallas kernel here")


# speedup vs baseline: 4.6499x; 4.6499x over previous
import functools

import jax
import jax.numpy as jnp
from jax import lax
from jax.experimental import pallas as pl
from jax.experimental.pallas import tpu as pltpu
from jax.experimental.pallas import tpu_sc as plsc

_EMB = 200
_EMBP = 256
_HID = 32
_B = 1024
_L = 200

_N_TOK = _B * _L
_CH = 128


def _gather_kernel(table_hbm, idx_hbm, out_hbm, idx_v, rows_v, sem, *, nch):
    nc = plsc.get_sparse_core_info().num_cores
    wid = lax.axis_index("s") * nc + lax.axis_index("c")
    base_row = wid * nch * _CH
    pltpu.sync_copy(idx_hbm.at[pl.ds(wid * nch, nch)], idx_v)

    def chunk(j, carry):
        pltpu.async_copy(table_hbm.at[idx_v.at[j, 0]], rows_v, sem).wait()
        off = pl.multiple_of(base_row + j * _CH, _CH)
        pltpu.sync_copy(rows_v, out_hbm.at[pl.ds(off, _CH)])
        return carry

    lax.fori_loop(0, nch, chunk, 0)


def _sc_gather(emb, ids_flat):
    info = plsc.get_sparse_core_info()
    nw = info.num_cores * info.num_subcores
    nch = _N_TOK // (nw * _CH)
    idx3d = ids_flat.reshape(_N_TOK // _CH, 1, _CH)
    mesh = plsc.VectorSubcoreMesh(core_axis_name="c", subcore_axis_name="s")
    f = pl.kernel(
        functools.partial(_gather_kernel, nch=nch),
        out_type=jax.ShapeDtypeStruct((_N_TOK, _EMBP), jnp.float32),
        mesh=mesh,
        scratch_types=[
            pltpu.VMEM((nch, 1, _CH), jnp.int32),
            pltpu.VMEM((_CH, _EMBP), jnp.float32),
            pltpu.SemaphoreType.DMA,
        ],
    )
    return f(emb, idx3d)


def _scan_kernel(xf_ref, xb_ref, wihf, wihb, whhf, whhb, bif, bhf, bib, bhb,
                 wout_t, bout_ref, o_ref, hf, hb):
    t = pl.program_id(0)

    @pl.when(t == 0)
    def _init():
        hf[...] = jnp.zeros_like(hf)
        hb[...] = jnp.zeros_like(hb)

    def step(x, h_ref, wih, whh, bi, bh):
        gi = jnp.dot(x, wih[...], preferred_element_type=jnp.float32) + bi[...]
        h = h_ref[...]
        gh = jnp.dot(h, whh[...], preferred_element_type=jnp.float32) + bh[...]
        r = jax.nn.sigmoid(gi[:, 0:_HID] + gh[:, 0:_HID])
        z = jax.nn.sigmoid(gi[:, _HID:2 * _HID] + gh[:, _HID:2 * _HID])
        n = jnp.tanh(gi[:, 2 * _HID:] + r * gh[:, 2 * _HID:])
        h_ref[...] = (1.0 - z) * n + z * h

    step(xf_ref[0], hf, wihf, whhf, bif, bhf)
    step(xb_ref[0], hb, wihb, whhb, bib, bhb)

    @pl.when(t == _L - 1)
    def _head():
        s_v = hf[...] + hb[...]
        raw = jnp.dot(s_v, wout_t[...], preferred_element_type=jnp.float32)
        o_ref[...] = jax.nn.sigmoid(raw + bout_ref[...])


def _tc_scan(x_all, wihf_t, wihb_t, whhf_t, whhb_t, bif, bhf, bib, bhb,
             wout_t, bout2):
    const = pl.BlockSpec(index_map=lambda t: (0, 0))
    return pl.pallas_call(
        _scan_kernel,
        grid=(_L,),
        in_specs=[
            pl.BlockSpec((1, _B, _EMBP), lambda t: (t, 0, 0)),
            pl.BlockSpec((1, _B, _EMBP), lambda t: (_L - 1 - t, 0, 0)),
            const, const, const, const, const, const, const, const,
            const, const,
        ],
        out_specs=pl.BlockSpec((_B, 1), lambda t: (0, 0)),
        out_shape=jax.ShapeDtypeStruct((_B, 1), jnp.float32),
        scratch_shapes=[
            pltpu.VMEM((_B, _HID), jnp.float32),
            pltpu.VMEM((_B, _HID), jnp.float32),
        ],
    )(x_all, x_all, wihf_t, wihb_t, whhf_t, whhb_t, bif, bhf, bib, bhb,
      wout_t, bout2)


def kernel(sentence_token, emb, Wih_f, Whh_f, bih_f, bhh_f,
           Wih_b, Whh_b, bih_b, bhh_b, Wout, bout):
    ids_flat = jnp.transpose(sentence_token).reshape(_N_TOK).astype(jnp.int32)
    emb_p = jnp.pad(emb, ((0, 0), (0, _EMBP - _EMB)))
    x_flat = _sc_gather(emb_p, ids_flat)
    x_all = x_flat.reshape(_L, _B, _EMBP)
    pad_w = ((0, _EMBP - _EMB), (0, 0))
    out = _tc_scan(
        x_all,
        jnp.pad(jnp.transpose(Wih_f), pad_w), jnp.pad(jnp.transpose(Wih_b), pad_w),
        jnp.transpose(Whh_f), jnp.transpose(Whh_b),
        bih_f[None, :], bhh_f[None, :], bih_b[None, :], bhh_b[None, :],
        jnp.transpose(Wout), bout[None, :],
    )
    return out


# speedup vs baseline: 6.7355x; 1.4485x over previous
import functools

import jax
import jax.numpy as jnp
from jax import lax
from jax.experimental import pallas as pl
from jax.experimental.pallas import tpu as pltpu
from jax.experimental.pallas import tpu_sc as plsc

_VOCAB = 100000
_EMB = 200
_GW = 256
_HID = 32
_B = 1024
_L = 200

_N_TOK = _B * _L
_CH = 128
_V_BLK = 2000


def _proj_kernel(emb_ref, w_ref, o_ref):
    o_ref[...] = jnp.dot(emb_ref[...], w_ref[...],
                         preferred_element_type=jnp.float32)


def _project(emb, wcat):
    return pl.pallas_call(
        _proj_kernel,
        grid=(_VOCAB // _V_BLK,),
        in_specs=[
            pl.BlockSpec((_V_BLK, _EMB), lambda i: (i, 0)),
            pl.BlockSpec(index_map=lambda i: (0, 0)),
        ],
        out_specs=pl.BlockSpec((_V_BLK, _GW), lambda i: (i, 0)),
        out_shape=jax.ShapeDtypeStruct((_VOCAB, _GW), jnp.float32),
    )(emb, wcat)


def _gather_kernel(table_hbm, idx_hbm, out_hbm, idx_v, rows_v, sem, *, nch):
    nc = plsc.get_sparse_core_info().num_cores
    wid = lax.axis_index("s") * nc + lax.axis_index("c")
    base_row = wid * nch * _CH
    pltpu.sync_copy(idx_hbm.at[pl.ds(wid * nch, nch)], idx_v)

    def chunk(j, carry):
        pltpu.async_copy(table_hbm.at[idx_v.at[j, 0]], rows_v, sem).wait()
        off = pl.multiple_of(base_row + j * _CH, _CH)
        pltpu.sync_copy(rows_v, out_hbm.at[pl.ds(off, _CH)])
        return carry

    lax.fori_loop(0, nch, chunk, 0)


def _sc_gather(table, ids_flat):
    info = plsc.get_sparse_core_info()
    nw = info.num_cores * info.num_subcores
    nch = _N_TOK // (nw * _CH)
    idx3d = ids_flat.reshape(_N_TOK // _CH, 1, _CH)
    mesh = plsc.VectorSubcoreMesh(core_axis_name="c", subcore_axis_name="s")
    f = pl.kernel(
        functools.partial(_gather_kernel, nch=nch),
        out_type=jax.ShapeDtypeStruct((_N_TOK, _GW), jnp.float32),
        mesh=mesh,
        scratch_types=[
            pltpu.VMEM((nch, 1, _CH), jnp.int32),
            pltpu.VMEM((_CH, _GW), jnp.float32),
            pltpu.SemaphoreType.DMA,
        ],
    )
    return f(table, idx3d)


def _scan_kernel(gif_ref, gib_ref, whhf, whhb, bif, bhf, bib, bhb,
                 wout_t, bout_ref, o_ref, hf, hb):
    t = pl.program_id(0)

    @pl.when(t == 0)
    def _init():
        hf[...] = jnp.zeros_like(hf)
        hb[...] = jnp.zeros_like(hb)

    def step(gi_blk, h_ref, whh, bi, bh):
        gi = gi_blk[:, 0:3 * _HID] + bi[...]
        h = h_ref[...]
        gh = jnp.dot(h, whh[...], preferred_element_type=jnp.float32) + bh[...]
        r = jax.nn.sigmoid(gi[:, 0:_HID] + gh[:, 0:_HID])
        z = jax.nn.sigmoid(gi[:, _HID:2 * _HID] + gh[:, _HID:2 * _HID])
        n = jnp.tanh(gi[:, 2 * _HID:3 * _HID] + r * gh[:, 2 * _HID:])
        h_ref[...] = (1.0 - z) * n + z * h

    step(gif_ref[0], hf, whhf, bif, bhf)
    step(gib_ref[0], hb, whhb, bib, bhb)

    @pl.when(t == _L - 1)
    def _head():
        s_v = hf[...] + hb[...]
        raw = jnp.dot(s_v, wout_t[...], preferred_element_type=jnp.float32)
        o_ref[...] = jax.nn.sigmoid(raw + bout_ref[...])


def _tc_scan(gi_all, whhf_t, whhb_t, bif, bhf, bib, bhb, wout_t, bout2):
    const = pl.BlockSpec(index_map=lambda t: (0, 0))
    return pl.pallas_call(
        _scan_kernel,
        grid=(_L,),
        in_specs=[
            pl.BlockSpec((1, _B, 128), lambda t: (t, 0, 0)),
            pl.BlockSpec((1, _B, 128), lambda t: (_L - 1 - t, 0, 1)),
            const, const, const, const, const, const, const, const,
        ],
        out_specs=pl.BlockSpec((_B, 1), lambda t: (0, 0)),
        out_shape=jax.ShapeDtypeStruct((_B, 1), jnp.float32),
        scratch_shapes=[
            pltpu.VMEM((_B, _HID), jnp.float32),
            pltpu.VMEM((_B, _HID), jnp.float32),
        ],
    )(gi_all, gi_all, whhf_t, whhb_t, bif, bhf, bib, bhb, wout_t, bout2)


def kernel(sentence_token, emb, Wih_f, Whh_f, bih_f, bhh_f,
           Wih_b, Whh_b, bih_b, bhh_b, Wout, bout):
    ids_flat = jnp.transpose(sentence_token).reshape(_N_TOK).astype(jnp.int32)
    wcat = jnp.zeros((_EMB, _GW), dtype=jnp.float32)
    wcat = wcat.at[:, 0:3 * _HID].set(jnp.transpose(Wih_f))
    wcat = wcat.at[:, 128:128 + 3 * _HID].set(jnp.transpose(Wih_b))
    table = _project(emb, wcat)
    gi_flat = _sc_gather(table, ids_flat)
    gi_all = gi_flat.reshape(_L, _B, _GW)
    out = _tc_scan(
        gi_all,
        jnp.transpose(Whh_f), jnp.transpose(Whh_b),
        bih_f[None, :], bhh_f[None, :], bih_b[None, :], bhh_b[None, :],
        jnp.transpose(Wout), bout[None, :],
    )
    return out
